# Initial kernel scaffold; baseline (speedup 1.0000x reference)
import jax
import jax.numpy as jnp
from jax.experimental import pallas as pl


def kernel(x_in, adj, emb, W1, a1, W2, a2, Wf, af, Wout, bout):
    raise NotImplementedError("write your pallas kernel here")


# speedup vs baseline: 2935.9235x; 2935.9235x over previous
import jax
import jax.numpy as jnp
from jax.experimental import pallas as pl

_N = 1024
_NHID = 32
_NHEADS = 4


def _layernorm(x, eps=1e-5):
    m = jnp.mean(x, axis=-1, keepdims=True)
    v = jnp.var(x, axis=-1, keepdims=True)
    return (x - m) / jnp.sqrt(v + eps)


def _leaky_relu(x, alpha=0.2):
    return jnp.where(x >= 0, x, alpha * x)


def _elu(x):
    return jnp.where(x > 0, x, jnp.exp(x) - 1.0)


def _gat_layer(x, mask, W_ref, a_ref, concat):
    outs = []
    ones = jnp.ones((_N, 1), dtype=jnp.float32)
    for i in range(_NHEADS):
        W = W_ref[i]
        a = a_ref[i, 0]
        h = jnp.dot(x, W, preferred_element_type=jnp.float32)
        e_src = jnp.sum(h * a[:_NHID][None, :], axis=1, keepdims=True)
        e_dst = jnp.sum(h * a[_NHID:][None, :], axis=1, keepdims=True)
        e = e_src + jnp.transpose(e_dst)
        E = jnp.exp(-_leaky_relu(e)) * mask
        h_aug = jnp.concatenate([h, ones], axis=1)
        nd = jnp.dot(E, h_aug, preferred_element_type=jnp.float32)
        hp = nd[:, :_NHID] / nd[:, _NHID:_NHID + 1]
        outs.append(_elu(hp) if concat else hp)
    return jnp.concatenate(outs, axis=1)


def _gat_body(x_in_ref, adj_ref, emb_ref, W1_ref, a1_ref, W2_ref, a2_ref,
              Wf_ref, af_ref, Wout_ref, bout_ref, out_ref):
    mask = adj_ref[...].astype(jnp.float32)
    x = jnp.dot(x_in_ref[...], emb_ref[...], preferred_element_type=jnp.float32)
    x = _layernorm(x)
    x = _layernorm(_gat_layer(x, mask, W1_ref, a1_ref, True))
    x = _layernorm(_gat_layer(x, mask, W2_ref, a2_ref, True))
    x = _layernorm(_gat_layer(x, mask, Wf_ref, af_ref, False))
    x = _elu(x)
    logits = jnp.dot(x, Wout_ref[...], preferred_element_type=jnp.float32)
    logits = logits + bout_ref[...][None, :]
    m = jnp.max(logits, axis=1, keepdims=True)
    s = logits - m
    lse = jnp.log(jnp.sum(jnp.exp(s), axis=1, keepdims=True))
    out_ref[...] = s - lse


def kernel(x_in, adj, emb, W1, a1, W2, a2, Wf, af, Wout, bout):
    return pl.pallas_call(
        _gat_body,
        out_shape=jax.ShapeDtypeStruct((_N, 40), jnp.float32),
    )(x_in, adj, emb, W1, a1, W2, a2, Wf, af, Wout, bout)
